# baseline (speedup 1.0000x reference)
import functools

import jax
import jax.numpy as jnp
from jax import lax
from jax.experimental import pallas as pl
from jax.experimental.pallas import tpu as pltpu
from jax.experimental.pallas import tpu_sc as plsc

N = 10000
E = 320000
D = 128
NC = 2
NS = 16
K = 80
NPAD = 10240
RPT = NPAD // NS
DEGW = 128

DEG_CHUNKS = E // (NS * K)
MSG_CHUNKS = E // (NC * NS * K)

_mesh = plsc.VectorSubcoreMesh(core_axis_name="c", subcore_axis_name="s")


def _deg_body(e_hbm, ones_hbm, z_hbm, out_hbm, idx_all, ones_v, acc_sp, sem,
              *, chunks):
    c = lax.axis_index("c")
    s = lax.axis_index("s")
    pltpu.sync_copy(e_hbm.at[c, s], idx_all)
    pltpu.sync_copy(ones_hbm, ones_v)
    pltpu.sync_copy(z_hbm, acc_sp.at[pl.ds(s * RPT, RPT)])
    plsc.subcore_barrier()

    FIRE = 10

    def scatter(j, _):
        for f in range(FIRE):
            pltpu.async_copy(ones_v, acc_sp.at[idx_all.at[j * FIRE + f]],
                             sem, add=True)
        for f in range(FIRE):
            pltpu.make_async_copy(ones_v, acc_sp.at[idx_all.at[j * FIRE + f]],
                                  sem).wait()
        return 0

    lax.fori_loop(0, chunks // FIRE, scatter, 0)
    plsc.subcore_barrier()
    pltpu.sync_copy(acc_sp.at[pl.ds(s * RPT, RPT)],
                    out_hbm.at[c, pl.ds(s * RPT, RPT)])


@jax.jit
def _deg_call(e_a, ones_a, z_a):
    return pl.kernel(
        functools.partial(_deg_body, chunks=DEG_CHUNKS),
        out_type=jax.ShapeDtypeStruct((NC, NPAD, DEGW), jnp.float32),
        mesh=_mesh,
        scratch_types=[
            pltpu.VMEM((DEG_CHUNKS, K), jnp.int32),
            pltpu.VMEM((K, DEGW), jnp.float32),
            pltpu.VMEM_SHARED((NPAD, DEGW), jnp.float32),
            pltpu.SemaphoreType.DMA,
        ],
    )(e_a, ones_a, z_a)


def _msg_body(h_hbm, src_hbm, dst_hbm, z_hbm, out_hbm,
              idx_s, idx_d, rows0, rows1, acc_sp, sem0, sem1, ses0, ses1,
              *, chunks):
    c = lax.axis_index("c")
    s = lax.axis_index("s")
    pltpu.sync_copy(src_hbm.at[c, s], idx_s)
    pltpu.sync_copy(dst_hbm.at[c, s], idx_d)
    pltpu.sync_copy(z_hbm, acc_sp.at[pl.ds(s * RPT, RPT)])
    plsc.subcore_barrier()

    def fire_g(j, buf, sem):
        pltpu.async_copy(h_hbm.at[idx_s.at[pl.ds(j * K, K)]], buf, sem)

    def wait_g(j, buf, sem):
        pltpu.make_async_copy(h_hbm.at[idx_s.at[pl.ds(j * K, K)]],
                              buf, sem).wait()

    def fire_s(j, buf, sem):
        pltpu.async_copy(buf, acc_sp.at[idx_d.at[j]], sem, add=True)

    def wait_s(j, buf, sem):
        pltpu.make_async_copy(buf, acc_sp.at[idx_d.at[j]], sem).wait()

    fire_g(0, rows0, sem0)
    fire_g(1, rows1, sem1)

    def pair(i, _):
        j0 = 2 * i
        wait_g(j0, rows0, sem0)
        fire_s(j0, rows0, ses0)
        wait_g(j0 + 1, rows1, sem1)
        fire_s(j0 + 1, rows1, ses1)
        wait_s(j0, rows0, ses0)
        fire_g(j0 + 2, rows0, sem0)
        wait_s(j0 + 1, rows1, ses1)
        fire_g(j0 + 3, rows1, sem1)
        return 0

    lax.fori_loop(0, (chunks - 3) // 2, pair, 0)
    jl = chunks - 3
    wait_g(jl, rows0, sem0)
    fire_s(jl, rows0, ses0)
    wait_g(jl + 1, rows1, sem1)
    fire_s(jl + 1, rows1, ses1)
    wait_s(jl, rows0, ses0)
    fire_g(jl + 2, rows0, sem0)
    wait_s(jl + 1, rows1, ses1)
    wait_g(jl + 2, rows0, sem0)
    fire_s(jl + 2, rows0, ses0)
    wait_s(jl + 2, rows0, ses0)
    plsc.subcore_barrier()
    pltpu.sync_copy(acc_sp.at[pl.ds(s * RPT, RPT)],
                    out_hbm.at[c, pl.ds(s * RPT, RPT)])


@jax.jit
def _msg_call(h, src_c, dst_c, z_m):
    return pl.kernel(
        functools.partial(_msg_body, chunks=MSG_CHUNKS),
        out_type=jax.ShapeDtypeStruct((NC, NPAD, D), jnp.float32),
        mesh=_mesh,
        scratch_types=[
            pltpu.VMEM((MSG_CHUNKS * K,), jnp.int32),
            pltpu.VMEM((MSG_CHUNKS, K), jnp.int32),
            pltpu.VMEM((K, D), jnp.float32),
            pltpu.VMEM((K, D), jnp.float32),
            pltpu.VMEM_SHARED((NPAD, D), jnp.float32),
            pltpu.SemaphoreType.DMA,
            pltpu.SemaphoreType.DMA,
            pltpu.SemaphoreType.DMA,
            pltpu.SemaphoreType.DMA,
        ],
    )(h, src_c, dst_c, z_m)


def _scale_body(x_ref, deg_ref, o_ref):
    d = deg_ref[:, 0:1]
    o_ref[...] = x_ref[...] * lax.rsqrt(jnp.maximum(d, 1.0))


@jax.jit
def _scale_call(x, deg_src):
    blk = 1000
    return pl.pallas_call(
        _scale_body,
        grid=(N // blk,),
        in_specs=[
            pl.BlockSpec((blk, D), lambda i: (i, 0)),
            pl.BlockSpec((blk, DEGW), lambda i: (i, 0)),
        ],
        out_specs=pl.BlockSpec((blk, D), lambda i: (i, 0)),
        out_shape=jax.ShapeDtypeStruct((N, D), jnp.float32),
    )(x, deg_src)


def _final_body(p0_ref, p1_ref, deg_ref, w_ref, b_ref, o_ref):
    norm = lax.rsqrt(jnp.maximum(deg_ref[:, 0:1], 1.0))
    agg = (p0_ref[...] + p1_ref[...]) * norm
    o_ref[...] = (jnp.dot(agg, w_ref[...], preferred_element_type=jnp.float32)
                  + b_ref[0, :])


@jax.jit
def _final_call(p0, p1, deg_dst, w, b8):
    blk = 1000
    return pl.pallas_call(
        _final_body,
        grid=(N // blk,),
        in_specs=[
            pl.BlockSpec((blk, D), lambda i: (i, 0)),
            pl.BlockSpec((blk, D), lambda i: (i, 0)),
            pl.BlockSpec((blk, DEGW), lambda i: (i, 0)),
            pl.BlockSpec((D, D), lambda i: (0, 0)),
            pl.BlockSpec((8, D), lambda i: (0, 0)),
        ],
        out_specs=pl.BlockSpec((blk, D), lambda i: (i, 0)),
        out_shape=jax.ShapeDtypeStruct((N, D), jnp.float32),
    )(p0, p1, deg_dst, w, b8)


def kernel(x, edge_index, W, b):
    ei = edge_index.astype(jnp.int32)
    e_a = ei.reshape(NC, NS, DEG_CHUNKS, K)
    ones_a = jnp.ones((K, DEGW), jnp.float32)
    z_a = jnp.zeros((RPT, DEGW), jnp.float32)
    degs = _deg_call(e_a, ones_a, z_a)
    h = _scale_call(x, degs[0])
    src_c = ei[0].reshape(NC, NS, MSG_CHUNKS * K)
    dst_c = ei[1].reshape(NC, NS, MSG_CHUNKS, K)
    z_m = jnp.zeros((RPT, D), jnp.float32)
    partials = _msg_call(h, src_c, dst_c, z_m)
    out = _final_call(partials[0], partials[1], degs[1], W,
                      jnp.broadcast_to(b, (8, D)))
    return out


# speedup vs baseline: 1.1311x; 1.1311x over previous
import functools

import jax
import jax.numpy as jnp
from jax import lax
from jax.experimental import pallas as pl
from jax.experimental.pallas import tpu as pltpu
from jax.experimental.pallas import tpu_sc as plsc

N = 10000
E = 320000
D = 128
NC = 2
NS = 16
K = 80
NPAD = 10240
RPT = NPAD // NS
DEGW = 128
DEGO = 16

DEG_CHUNKS = E // (NS * K)
MSG_CHUNKS = E // (NC * NS * K)


_mesh = plsc.VectorSubcoreMesh(core_axis_name="c", subcore_axis_name="s")


def _deg_body(e_hbm, ones_hbm, z_hbm, out_hbm, idx_all, ones_v, acc_sp, sem,
              *, chunks):
    c = lax.axis_index("c")
    s = lax.axis_index("s")
    pltpu.sync_copy(e_hbm.at[c, s], idx_all)
    pltpu.sync_copy(ones_hbm, ones_v)
    pltpu.sync_copy(z_hbm, acc_sp.at[pl.ds(s * RPT, RPT)])
    plsc.subcore_barrier()

    FIRE = 10

    def scatter(j, _):
        for f in range(FIRE):
            pltpu.async_copy(ones_v, acc_sp.at[idx_all.at[j * FIRE + f]],
                             sem, add=True)
        for f in range(FIRE):
            pltpu.make_async_copy(ones_v, acc_sp.at[idx_all.at[j * FIRE + f]],
                                  sem).wait()
        return 0

    lax.fori_loop(0, chunks // FIRE, scatter, 0)
    plsc.subcore_barrier()
    pltpu.sync_copy(acc_sp.at[pl.ds(s * RPT, RPT)],
                    out_hbm.at[c, pl.ds(s * RPT, RPT)])


@jax.jit
def _deg_call(e_a, ones_a, z_a):
    return pl.kernel(
        functools.partial(_deg_body, chunks=DEG_CHUNKS),
        out_type=jax.ShapeDtypeStruct((NC, NPAD, DEGW), jnp.float32),
        mesh=_mesh,
        scratch_types=[
            pltpu.VMEM((DEG_CHUNKS, K), jnp.int32),
            pltpu.VMEM((K, DEGW), jnp.float32),
            pltpu.VMEM_SHARED((NPAD, DEGW), jnp.float32),
            pltpu.SemaphoreType.DMA,
        ],
    )(e_a, ones_a, z_a)


def _msg_body(h_hbm, src_hbm, dst_hbm, z_hbm, out_hbm,
              idx_s, idx_d, rows0, rows1, acc_sp, sem0, sem1, *, chunks):
    c = lax.axis_index("c")
    s = lax.axis_index("s")
    pltpu.sync_copy(src_hbm.at[c, s], idx_s)
    pltpu.sync_copy(dst_hbm.at[c, s], idx_d)
    pltpu.sync_copy(z_hbm, acc_sp.at[pl.ds(s * RPT, RPT)])
    plsc.subcore_barrier()

    def fire(j, buf, sem):
        pltpu.async_copy(h_hbm.at[idx_s.at[pl.ds(j * K, K)]], buf, sem)

    def drain(j, buf, sem):
        pltpu.make_async_copy(h_hbm.at[idx_s.at[pl.ds(j * K, K)]],
                              buf, sem).wait()

    def scat(j, buf):
        pltpu.sync_copy(buf, acc_sp.at[idx_d.at[j]], add=True)

    fire(0, rows0, sem0)

    def pair(i, _):
        j0 = 2 * i
        fire(j0 + 1, rows1, sem1)
        drain(j0, rows0, sem0)
        scat(j0, rows0)
        fire(j0 + 2, rows0, sem0)
        drain(j0 + 1, rows1, sem1)
        scat(j0 + 1, rows1)
        return 0

    lax.fori_loop(0, (chunks - 1) // 2, pair, 0)
    drain(chunks - 1, rows0, sem0)
    scat(chunks - 1, rows0)
    plsc.subcore_barrier()
    pltpu.sync_copy(acc_sp.at[pl.ds(s * RPT, RPT)],
                    out_hbm.at[c, pl.ds(s * RPT, RPT)])


@jax.jit
def _msg_call(h, src_c, dst_c, z_m):
    return pl.kernel(
        functools.partial(_msg_body, chunks=MSG_CHUNKS),
        out_type=jax.ShapeDtypeStruct((NC, NPAD, D), jnp.float32),
        mesh=_mesh,
        scratch_types=[
            pltpu.VMEM((MSG_CHUNKS * K,), jnp.int32),
            pltpu.VMEM((MSG_CHUNKS, K), jnp.int32),
            pltpu.VMEM((K, D), jnp.float32),
            pltpu.VMEM((K, D), jnp.float32),
            pltpu.VMEM_SHARED((NPAD, D), jnp.float32),
            pltpu.SemaphoreType.DMA,
            pltpu.SemaphoreType.DMA,
        ],
    )(h, src_c, dst_c, z_m)


def _scale_body(x_ref, deg_ref, o_ref):
    d = deg_ref[:, 0:1]
    o_ref[...] = x_ref[...] * lax.rsqrt(jnp.maximum(d, 1.0))


@jax.jit
def _scale_call(x, deg_src):
    blk = 2000
    return pl.pallas_call(
        _scale_body,
        grid=(N // blk,),
        in_specs=[
            pl.BlockSpec((blk, D), lambda i: (i, 0)),
            pl.BlockSpec((blk, DEGW), lambda i: (i, 0)),
        ],
        out_specs=pl.BlockSpec((blk, D), lambda i: (i, 0)),
        out_shape=jax.ShapeDtypeStruct((N, D), jnp.float32),
    )(x, deg_src)


def _final_body(p0_ref, p1_ref, deg_ref, w_ref, b_ref, o_ref):
    norm = lax.rsqrt(jnp.maximum(deg_ref[:, 0:1], 1.0))
    agg = (p0_ref[...] + p1_ref[...]) * norm
    o_ref[...] = (jnp.dot(agg, w_ref[...], preferred_element_type=jnp.float32)
                  + b_ref[0, :])


@jax.jit
def _final_call(p0, p1, deg_dst, w, b8):
    blk = 2000
    return pl.pallas_call(
        _final_body,
        grid=(N // blk,),
        in_specs=[
            pl.BlockSpec((blk, D), lambda i: (i, 0)),
            pl.BlockSpec((blk, D), lambda i: (i, 0)),
            pl.BlockSpec((blk, DEGW), lambda i: (i, 0)),
            pl.BlockSpec((D, D), lambda i: (0, 0)),
            pl.BlockSpec((8, D), lambda i: (0, 0)),
        ],
        out_specs=pl.BlockSpec((blk, D), lambda i: (i, 0)),
        out_shape=jax.ShapeDtypeStruct((N, D), jnp.float32),
    )(p0, p1, deg_dst, w, b8)


def kernel(x, edge_index, W, b):
    ei = edge_index.astype(jnp.int32)
    e_a = ei.reshape(NC, NS, DEG_CHUNKS, K)
    ones_a = jnp.ones((K, DEGW), jnp.float32)
    z_a = jnp.zeros((RPT, DEGW), jnp.float32)
    degs = _deg_call(e_a, ones_a, z_a)
    h = _scale_call(x, degs[0])
    src_c = ei[0].reshape(NC, NS, MSG_CHUNKS * K)
    dst_c = ei[1].reshape(NC, NS, MSG_CHUNKS, K)
    z_m = jnp.zeros((RPT, D), jnp.float32)
    partials = _msg_call(h, src_c, dst_c, z_m)
    out = _final_call(partials[0], partials[1], degs[1], W,
                      jnp.broadcast_to(b, (8, D)))
    return out


# speedup vs baseline: 1.1330x; 1.0017x over previous
import functools

import jax
import jax.numpy as jnp
from jax import lax
from jax.experimental import pallas as pl
from jax.experimental.pallas import tpu as pltpu
from jax.experimental.pallas import tpu_sc as plsc

N = 10000
E = 320000
D = 128
NC = 2
NS = 16
K = 80
NPAD = 10240
RPT = NPAD // NS
DEGW = 128

DEG_CHUNKS = E // (NS * K)
MSG_CHUNKS = E // (NC * NS * K)


_mesh = plsc.VectorSubcoreMesh(core_axis_name="c", subcore_axis_name="s",
                               num_cores=NC, num_subcores=NS)


def _deg_body(e_hbm, ones_hbm, z_hbm, out_hbm, idx_all, ones_v, acc_sp, sem,
              *, chunks):
    c = lax.axis_index("c")
    s = lax.axis_index("s")
    pltpu.sync_copy(e_hbm.at[c, s], idx_all)
    pltpu.sync_copy(ones_hbm, ones_v)
    pltpu.sync_copy(z_hbm, acc_sp.at[pl.ds(s * RPT, RPT)])
    plsc.subcore_barrier()

    FIRE = 10

    def scatter(j, _):
        for f in range(FIRE):
            pltpu.async_copy(ones_v, acc_sp.at[idx_all.at[j * FIRE + f]],
                             sem, add=True)
        for f in range(FIRE):
            pltpu.make_async_copy(ones_v, acc_sp.at[idx_all.at[j * FIRE + f]],
                                  sem).wait()
        return 0

    lax.fori_loop(0, chunks // FIRE, scatter, 0)
    plsc.subcore_barrier()
    pltpu.sync_copy(acc_sp.at[pl.ds(s * RPT, RPT)],
                    out_hbm.at[c, pl.ds(s * RPT, RPT)])


@jax.jit
def _deg_call(e_a, ones_a, z_a):
    return pl.kernel(
        functools.partial(_deg_body, chunks=DEG_CHUNKS),
        out_type=jax.ShapeDtypeStruct((NC, NPAD, DEGW), jnp.float32),
        mesh=_mesh,
        scratch_types=[
            pltpu.VMEM((DEG_CHUNKS, K), jnp.int32),
            pltpu.VMEM((K, DEGW), jnp.float32),
            pltpu.VMEM_SHARED((NPAD, DEGW), jnp.float32),
            pltpu.SemaphoreType.DMA,
        ],
    )(e_a, ones_a, z_a)


def _msg_body(h_hbm, src_hbm, dst_hbm, z_hbm, out_hbm,
              idx_s, idx_d, rows0, rows1, acc_sp, sem0, sem1, *, chunks):
    c = lax.axis_index("c")
    s = lax.axis_index("s")
    pltpu.sync_copy(src_hbm.at[c, s], idx_s)
    pltpu.sync_copy(dst_hbm.at[c, s], idx_d)
    pltpu.sync_copy(z_hbm, acc_sp.at[pl.ds(s * RPT, RPT)])
    plsc.subcore_barrier()

    def fire(j, buf, sem):
        pltpu.async_copy(h_hbm.at[idx_s.at[pl.ds(j * K, K)]], buf, sem)

    def drain(j, buf, sem):
        pltpu.make_async_copy(h_hbm.at[idx_s.at[pl.ds(j * K, K)]],
                              buf, sem).wait()

    def scat(j, buf):
        pltpu.sync_copy(buf, acc_sp.at[idx_d.at[j]], add=True)

    fire(0, rows0, sem0)

    def pair(i, _):
        j0 = 2 * i
        fire(j0 + 1, rows1, sem1)
        drain(j0, rows0, sem0)
        scat(j0, rows0)
        fire(j0 + 2, rows0, sem0)
        drain(j0 + 1, rows1, sem1)
        scat(j0 + 1, rows1)
        return 0

    lax.fori_loop(0, (chunks - 1) // 2, pair, 0)
    drain(chunks - 1, rows0, sem0)
    scat(chunks - 1, rows0)
    plsc.subcore_barrier()
    pltpu.sync_copy(acc_sp.at[pl.ds(s * RPT, RPT)],
                    out_hbm.at[c, pl.ds(s * RPT, RPT)])


@jax.jit
def _msg_call(h, src_c, dst_c, z_m):
    return pl.kernel(
        functools.partial(_msg_body, chunks=MSG_CHUNKS),
        out_type=jax.ShapeDtypeStruct((NC, NPAD, D), jnp.float32),
        mesh=_mesh,
        scratch_types=[
            pltpu.VMEM((MSG_CHUNKS * K,), jnp.int32),
            pltpu.VMEM((MSG_CHUNKS, K), jnp.int32),
            pltpu.VMEM((K, D), jnp.float32),
            pltpu.VMEM((K, D), jnp.float32),
            pltpu.VMEM_SHARED((NPAD, D), jnp.float32),
            pltpu.SemaphoreType.DMA,
            pltpu.SemaphoreType.DMA,
        ],
    )(h, src_c, dst_c, z_m)


def _scale_body(x_ref, deg_ref, o_ref):
    d = deg_ref[:, 0:1]
    o_ref[...] = x_ref[...] * lax.rsqrt(jnp.maximum(d, 1.0))


@jax.jit
def _scale_call(x, deg_src):
    blk = 2000
    return pl.pallas_call(
        _scale_body,
        grid=(N // blk,),
        in_specs=[
            pl.BlockSpec((blk, D), lambda i: (i, 0)),
            pl.BlockSpec((blk, DEGW), lambda i: (i, 0)),
        ],
        out_specs=pl.BlockSpec((blk, D), lambda i: (i, 0)),
        out_shape=jax.ShapeDtypeStruct((N, D), jnp.float32),
    )(x, deg_src)


def _final_body(p0_ref, p1_ref, deg_ref, w_ref, b_ref, o_ref):
    norm = lax.rsqrt(jnp.maximum(deg_ref[:, 0:1], 1.0))
    agg = (p0_ref[...] + p1_ref[...]) * norm
    o_ref[...] = (jnp.dot(agg, w_ref[...], preferred_element_type=jnp.float32)
                  + b_ref[0, :])


@jax.jit
def _final_call(p0, p1, deg_dst, w, b8):
    blk = 2000
    return pl.pallas_call(
        _final_body,
        grid=(N // blk,),
        in_specs=[
            pl.BlockSpec((blk, D), lambda i: (i, 0)),
            pl.BlockSpec((blk, D), lambda i: (i, 0)),
            pl.BlockSpec((blk, DEGW), lambda i: (i, 0)),
            pl.BlockSpec((D, D), lambda i: (0, 0)),
            pl.BlockSpec((8, D), lambda i: (0, 0)),
        ],
        out_specs=pl.BlockSpec((blk, D), lambda i: (i, 0)),
        out_shape=jax.ShapeDtypeStruct((N, D), jnp.float32),
    )(p0, p1, deg_dst, w, b8)


def kernel(x, edge_index, W, b):
    ei = edge_index.astype(jnp.int32)
    e_a = ei.reshape(NC, NS, DEG_CHUNKS, K)
    ones_a = jnp.ones((K, DEGW), jnp.float32)
    z_a = jnp.zeros((RPT, DEGW), jnp.float32)
    degs = _deg_call(e_a, ones_a, z_a)
    h = _scale_call(x, degs[0])
    src_c = ei[0].reshape(NC, NS, MSG_CHUNKS * K)
    dst_c = ei[1].reshape(NC, NS, MSG_CHUNKS, K)
    z_m = jnp.zeros((RPT, D), jnp.float32)
    partials = _msg_call(h, src_c, dst_c, z_m)
    out = _final_call(partials[0], partials[1], degs[1], W,
                      jnp.broadcast_to(b, (8, D)))
    return out
